# baseline (speedup 1.0000x reference)
import functools

import jax
import jax.numpy as jnp
from jax import lax
from jax.experimental import pallas as pl
from jax.experimental.pallas import tpu as pltpu
from jax.experimental.pallas import tpu_sc as plsc

NUM_AUTHORS = 50000
SPAN = 50000
DIM = 128
NW = 32
CHUNKS_PER_W = 4
NCHUNKS = NW * CHUNKS_PER_W
CH = -(-SPAN // (NCHUNKS * 8)) * 8

_mesh = plsc.VectorSubcoreMesh(core_axis_name="c", subcore_axis_name="s")


@functools.partial(
    pl.kernel,
    out_type=jax.ShapeDtypeStruct((SPAN, DIM), jnp.float32),
    mesh=_mesh,
    scratch_types=[pltpu.VMEM((16,), jnp.int32)],
)
def _sc_slice_copy(start_hbm, table_hbm, out_hbm, start_v):
    wid = lax.axis_index("s") * 2 + lax.axis_index("c")
    pltpu.sync_copy(start_hbm, start_v)
    srow = start_v[...][0]
    for j in range(CHUNKS_PER_W):
        c = wid * CHUNKS_PER_W + j
        r = pl.multiple_of(jnp.minimum(c * CH, SPAN - CH), 8)
        pltpu.sync_copy(
            table_hbm.at[pl.ds(pl.multiple_of(srow + r, 8), CH)],
            out_hbm.at[pl.ds(r, CH)],
        )


def kernel(node_type, embedding_weight):
    start = jnp.asarray([0, NUM_AUTHORS], dtype=jnp.int32)[node_type]
    start_vec = jnp.full((16,), start, dtype=jnp.int32)
    return _sc_slice_copy(start_vec, embedding_weight)


# speedup vs baseline: 20.2830x; 20.2830x over previous
import functools

import jax
import jax.numpy as jnp
from jax import lax
from jax.experimental import pallas as pl
from jax.experimental.pallas import tpu as pltpu
from jax.experimental.pallas import tpu_sc as plsc

NUM_AUTHORS = 50000
SPAN = 50000
DIM = 128
NW = 32
CHUNKS_PER_W = 4
NCHUNKS = NW * CHUNKS_PER_W
CH = -(-SPAN // (NCHUNKS * 8)) * 8

_mesh = plsc.VectorSubcoreMesh(core_axis_name="c", subcore_axis_name="s")


@functools.partial(
    pl.kernel,
    out_type=jax.ShapeDtypeStruct((SPAN, DIM), jnp.float32),
    mesh=_mesh,
    scratch_types=[
        pltpu.VMEM((16,), jnp.int32),
        pltpu.VMEM((CH, DIM), jnp.float32),
        pltpu.VMEM((CH, DIM), jnp.float32),
        pltpu.SemaphoreType.DMA,
        pltpu.SemaphoreType.DMA,
        pltpu.SemaphoreType.DMA,
        pltpu.SemaphoreType.DMA,
    ],
)
def _sc_slice_copy(start_hbm, table_hbm, out_hbm, start_v, b0, b1,
                   rs0, rs1, ws0, ws1):
    wid = lax.axis_index("s") * 2 + lax.axis_index("c")
    pltpu.sync_copy(start_hbm, start_v)
    srow = start_v[...][0]

    bufs = (b0, b1)
    rsems = (rs0, rs1)
    wsems = (ws0, ws1)

    def row(j):
        c = wid * CHUNKS_PER_W + j
        return pl.multiple_of(jnp.minimum(c * CH, SPAN - CH), 8)

    def start_read(j):
        return pltpu.async_copy(
            table_hbm.at[pl.ds(pl.multiple_of(srow + row(j), 8), CH)],
            bufs[j % 2], rsems[j % 2])

    def start_write(j):
        return pltpu.async_copy(
            bufs[j % 2], out_hbm.at[pl.ds(row(j), CH)], wsems[j % 2])

    reads = [None, None]
    writes = [None, None]
    for j in range(CHUNKS_PER_W):
        b = j % 2
        if writes[b] is not None:
            writes[b].wait()
        reads[b] = start_read(j)
        if j >= 1:
            pb = (j - 1) % 2
            reads[pb].wait()
            writes[pb] = start_write(j - 1)
    last = CHUNKS_PER_W - 1
    reads[last % 2].wait()
    writes[last % 2] = start_write(last)
    writes[0].wait()
    writes[1].wait()


def kernel(node_type, embedding_weight):
    start = jnp.asarray([0, NUM_AUTHORS], dtype=jnp.int32)[node_type]
    start_vec = jnp.full((16,), start, dtype=jnp.int32)
    return _sc_slice_copy(start_vec, embedding_weight)


# speedup vs baseline: 20.3994x; 1.0057x over previous
import functools

import jax
import jax.numpy as jnp
from jax import lax
from jax.experimental import pallas as pl
from jax.experimental.pallas import tpu as pltpu
from jax.experimental.pallas import tpu_sc as plsc

NUM_AUTHORS = 50000
SPAN = 50000
DIM = 128
NW = 32
CHUNKS_PER_W = 7
NBUF = 3
NCHUNKS = NW * CHUNKS_PER_W
CH = -(-SPAN // (NCHUNKS * 8)) * 8

_mesh = plsc.VectorSubcoreMesh(core_axis_name="c", subcore_axis_name="s")


@functools.partial(
    pl.kernel,
    out_type=jax.ShapeDtypeStruct((SPAN, DIM), jnp.float32),
    mesh=_mesh,
    scratch_types=(
        [pltpu.VMEM((16,), jnp.int32)]
        + [pltpu.VMEM((CH, DIM), jnp.float32) for _ in range(NBUF)]
        + [pltpu.SemaphoreType.DMA for _ in range(2 * NBUF)]
    ),
)
def _sc_slice_copy(start_hbm, table_hbm, out_hbm, start_v, *scratch):
    bufs = scratch[:NBUF]
    rsems = scratch[NBUF:2 * NBUF]
    wsems = scratch[2 * NBUF:]
    wid = lax.axis_index("s") * 2 + lax.axis_index("c")
    pltpu.sync_copy(start_hbm, start_v)
    srow = start_v[...][0]

    def row(j):
        c = wid * CHUNKS_PER_W + j
        return pl.multiple_of(jnp.minimum(c * CH, SPAN - CH), 8)

    def start_read(j, b):
        return pltpu.async_copy(
            table_hbm.at[pl.ds(pl.multiple_of(srow + row(j), 8), CH)],
            bufs[b], rsems[b])

    def start_write(j, b):
        return pltpu.async_copy(
            bufs[b], out_hbm.at[pl.ds(row(j), CH)], wsems[b])

    reads = [None] * NBUF
    writes = [None] * NBUF
    for j in range(CHUNKS_PER_W + 1):
        if j < CHUNKS_PER_W:
            b = j % NBUF
            if writes[b] is not None:
                writes[b].wait()
            reads[b] = start_read(j, b)
        if j >= 1:
            pb = (j - 1) % NBUF
            reads[pb].wait()
            writes[pb] = start_write(j - 1, pb)
    for b in range(NBUF):
        if writes[b] is not None:
            writes[b].wait()


def kernel(node_type, embedding_weight):
    start = jnp.asarray([0, NUM_AUTHORS], dtype=jnp.int32)[node_type]
    start_vec = jnp.full((16,), start, dtype=jnp.int32)
    return _sc_slice_copy(start_vec, embedding_weight)


# speedup vs baseline: 21.3262x; 1.0454x over previous
import functools

import jax
import jax.numpy as jnp
from jax import lax
from jax.experimental import pallas as pl
from jax.experimental.pallas import tpu as pltpu
from jax.experimental.pallas import tpu_sc as plsc

NUM_AUTHORS = 50000
SPAN = 50000
DIM = 128
NW = 32
NSUB = 16
CHUNKS_PER_W = 7
NBUF = 2
NCHUNKS = NW * CHUNKS_PER_W
CH = -(-SPAN // (NCHUNKS * 8)) * 8

_mesh = plsc.VectorSubcoreMesh(core_axis_name="c", subcore_axis_name="s")


@functools.partial(
    pl.kernel,
    out_type=jax.ShapeDtypeStruct((SPAN, DIM), jnp.float32),
    mesh=_mesh,
    scratch_types=(
        [pltpu.VMEM((16,), jnp.int32)]
        + [pltpu.VMEM((CH, DIM), jnp.float32) for _ in range(NBUF)]
        + [pltpu.VMEM_SHARED((NSUB, NBUF, CH, DIM), jnp.float32)]
        + [pltpu.SemaphoreType.DMA for _ in range(4 * NBUF)]
    ),
)
def _sc_slice_copy(start_hbm, table_hbm, out_hbm, start_v, *scratch):
    tbufs = scratch[:NBUF]
    shared = scratch[NBUF]
    sems = scratch[NBUF + 1:]
    wid = lax.axis_index("s") * 2 + lax.axis_index("c")
    sid = lax.axis_index("s")
    pltpu.sync_copy(start_hbm, start_v)
    srow = start_v[...][0]

    def row(j):
        c = wid * CHUNKS_PER_W + j
        return pl.multiple_of(jnp.minimum(c * CH, SPAN - CH), 8)

    def buf(path, b):
        return tbufs[b] if path == 0 else shared.at[sid, b]

    def start_read(j, path, b):
        return pltpu.async_copy(
            table_hbm.at[pl.ds(pl.multiple_of(srow + row(j), 8), CH)],
            buf(path, b), sems[4 * b + 2 * path])

    def start_write(j, path, b):
        return pltpu.async_copy(
            buf(path, b), out_hbm.at[pl.ds(row(j), CH)],
            sems[4 * b + 2 * path + 1])

    reads = {}
    writes = {}
    done = set()
    for j in range(CHUNKS_PER_W + 1):
        if j < CHUNKS_PER_W:
            prev = j - 2 * NBUF
            if prev >= 0:
                writes[prev].wait()
                done.add(prev)
            reads[j] = start_read(j, j % 2, (j // 2) % NBUF)
        if j >= 1:
            k = j - 1
            reads[k].wait()
            writes[k] = start_write(k, k % 2, (k // 2) % NBUF)
    for j in range(CHUNKS_PER_W):
        if j not in done:
            writes[j].wait()


def kernel(node_type, embedding_weight):
    start = jnp.asarray([0, NUM_AUTHORS], dtype=jnp.int32)[node_type]
    start_vec = jnp.full((16,), start, dtype=jnp.int32)
    return _sc_slice_copy(start_vec, embedding_weight)
